# baseline (speedup 1.0000x reference)
import functools

import jax
import jax.numpy as jnp
from jax import lax
from jax.experimental import pallas as pl
from jax.experimental.pallas import tpu as pltpu
from jax.experimental.pallas import tpu_sc as plsc

N_NODES = 50000
E_EDGES = 800000

NC = 2
NS = 16
CHUNK = 128

E_PAD = 802816
SPMM_STREAMS = E_PAD // NS // CHUNK

N_PAD = 50176
STRIPE = N_PAD // NS
ZROWS = 32
N4 = 4 * N_PAD

SEGS = 4
BLK = 1024
GRID = N_PAD // BLK


_SC_MESH = dict(core_axis_name="c", subcore_axis_name="s",
                num_cores=NC, num_subcores=NS)
_SC_PARAMS = pltpu.CompilerParams(use_tc_tiling_on_sc=False,
                                  needs_layout_passes=False)


def _fill(buf, rows, width, value):
    def body(i, carry):
        for w0 in range(0, width, 16):
            buf[i, pl.ds(w0, 16)] = jnp.full((16,), value, jnp.float32)
        return carry
    lax.fori_loop(0, rows, body, 0)


def _rsqrt16(v):
    i = plsc.bitcast(v, jnp.int32)
    i = 0x5F3759DF - lax.shift_right_logical(i, 1)
    y = plsc.bitcast(i, jnp.float32)
    for _ in range(3):
        y = y * (1.5 - 0.5 * v * y * y)
    return y


def _dinv_kernel_body(dst_hbm, dv_hbm, acc, dst_st, onesv, cbuf, obuf, ssem,
                      wsem):
    c = lax.axis_index("c")
    s = lax.axis_index("s")
    stripe0 = s * STRIPE
    nmac = SPMM_STREAMS // SEGS

    _fill(onesv, CHUNK, 16, 1.0)
    def idma(k, carry):
        pltpu.sync_copy(onesv, acc.at[pl.ds(stripe0 + k * CHUNK, CHUNK)])
        return carry
    lax.fori_loop(0, STRIPE // CHUNK, idma, 0)
    rem = STRIPE % CHUNK
    if rem:
        pltpu.sync_copy(onesv.at[pl.ds(0, rem)],
                        acc.at[pl.ds(stripe0 + STRIPE - rem, rem)])
    plsc.subcore_barrier()

    def stage(d, par):
        pltpu.async_copy(dst_hbm.at[s, pl.ds(d * SEGS, SEGS)],
                         dst_st.at[par], ssem)

    def stage_wait(d, par):
        pltpu.make_async_copy(dst_hbm.at[s, pl.ds(d * SEGS, SEGS)],
                              dst_st.at[par], ssem).wait()

    stage(0, 0)

    def step(d, carry):
        par = lax.rem(d, 2)
        stage_wait(d, par)

        @pl.when(d > 0)
        def _():
            pltpu.make_async_copy(onesv, acc.at[dst_st.at[par, 0]],
                                  wsem).wait()

        @pl.when(d + 1 < nmac)
        def _():
            stage(d + 1, 1 - par)

        for j in range(SEGS):
            if j > 0:
                pltpu.make_async_copy(onesv, acc.at[dst_st.at[par, 0]],
                                      wsem).wait()
            pltpu.async_copy(onesv, acc.at[dst_st.at[par, j]], wsem, add=True)
        return carry
    lax.fori_loop(0, nmac, step, 0)
    pltpu.make_async_copy(onesv, acc.at[pl.ds(0, CHUNK)], wsem).wait()
    plsc.subcore_barrier()

    half = STRIPE // NC
    base = stripe0 + c * half

    def flush(k, carry):
        pltpu.sync_copy(acc.at[pl.ds(base + k * 32, 32)], cbuf)
        for j in range(32):
            v = _rsqrt16(cbuf[j, pl.ds(0, 16)])
            for t in range(8):
                obuf[j, pl.ds(16 * t, 16)] = v
        pltpu.sync_copy(obuf, dv_hbm.at[pl.ds(base + k * 32, 32)])
        return carry
    lax.fori_loop(0, half // 32, flush, 0)


@functools.lru_cache(maxsize=None)
def _dinv_kernel():
    return functools.partial(
        pl.kernel,
        out_type=jax.ShapeDtypeStruct((N_PAD, 128), jnp.float32),
        mesh=plsc.VectorSubcoreMesh(**_SC_MESH),
        compiler_params=_SC_PARAMS,
        scratch_types=[
            pltpu.VMEM_SHARED((N_PAD, 16), jnp.float32),
            pltpu.VMEM((2, SEGS, CHUNK), jnp.int32),
            pltpu.VMEM((CHUNK, 16), jnp.float32),
            pltpu.VMEM((32, 16), jnp.float32),
            pltpu.VMEM((32, 128), jnp.float32),
            pltpu.SemaphoreType.DMA,
            pltpu.SemaphoreType.DMA,
        ],
    )(_dinv_kernel_body)


@functools.lru_cache(maxsize=None)
def _make_spmm(rounds, base=0):
    nmac = SPMM_STREAMS // SEGS

    def body(y4_hbm, src_hbm, dst_hbm, out_hbm,
             acc, src_st, dst_st, rows, zbuf, ssem, gsem, wsem):
        c = lax.axis_index("c")
        s = lax.axis_index("s")
        stripe0 = s * STRIPE

        _fill(zbuf, ZROWS, 32, 0.0)

        def stage(d, par):
            pltpu.async_copy(src_hbm.at[s, pl.ds(d * SEGS, SEGS)],
                             src_st.at[par], ssem)
            pltpu.async_copy(dst_hbm.at[s, pl.ds(d * SEGS, SEGS)],
                             dst_st.at[par], ssem)

        def stage_wait(d, par):
            pltpu.make_async_copy(src_hbm.at[s, pl.ds(d * SEGS, SEGS)],
                                  src_st.at[par], ssem).wait()
            pltpu.make_async_copy(dst_hbm.at[s, pl.ds(d * SEGS, SEGS)],
                                  dst_st.at[par], ssem).wait()

        def run_round(slab, last):
            col0 = 32 * slab
            table = y4_hbm.at[pl.ds(base + slab, N4 - 3 - base)]

            def zdma(k, carry):
                pltpu.sync_copy(zbuf, acc.at[pl.ds(stripe0 + k * ZROWS, ZROWS)])
                return carry
            lax.fori_loop(0, STRIPE // ZROWS, zdma, 0)
            plsc.subcore_barrier()

            stage(0, 0)

            def step(d, carry):
                par = lax.rem(d, 2)
                stage_wait(d, par)

                @pl.when(d > 0)
                def _():
                    pltpu.make_async_copy(
                        rows.at[0], acc.at[dst_st.at[par, 0]], wsem).wait()

                @pl.when(d + 1 < nmac)
                def _():
                    stage(d + 1, 1 - par)

                for j in range(SEGS):
                    pltpu.async_copy(table.at[src_st.at[par, j]],
                                     rows.at[j], gsem.at[j])
                for j in range(SEGS):
                    pltpu.make_async_copy(table.at[src_st.at[par, j]],
                                          rows.at[j], gsem.at[j]).wait()
                    if j > 0:
                        pltpu.make_async_copy(
                            rows.at[0], acc.at[dst_st.at[par, 0]], wsem).wait()
                    pltpu.async_copy(rows.at[j], acc.at[dst_st.at[par, j]],
                                     wsem, add=True)
                return carry
            lax.fori_loop(0, nmac, step, 0)

            pltpu.make_async_copy(rows.at[0], acc.at[pl.ds(0, CHUNK)],
                                  wsem).wait()
            plsc.subcore_barrier()

            pltpu.sync_copy(acc.at[pl.ds(stripe0, STRIPE)],
                            out_hbm.at[pl.ds(stripe0, STRIPE),
                                       pl.ds(col0, 32)])
            if not last:
                plsc.subcore_barrier()

        for r in range(rounds):
            for cc in range(NC):
                @pl.when(c == cc)
                def _(r=r, cc=cc):
                    run_round(2 * r + cc, r + 1 == rounds)

    return functools.partial(
        pl.kernel,
        out_type=jax.ShapeDtypeStruct((N_PAD, 128), jnp.float32),
        mesh=plsc.VectorSubcoreMesh(**_SC_MESH),
        compiler_params=_SC_PARAMS,
        scratch_types=[
            pltpu.VMEM_SHARED((N_PAD, 32), jnp.float32),
            pltpu.VMEM((2, SEGS, CHUNK), jnp.int32),
            pltpu.VMEM((2, SEGS, CHUNK), jnp.int32),
            pltpu.VMEM((SEGS, CHUNK, 32), jnp.float32),
            pltpu.VMEM((ZROWS, 32), jnp.float32),
            pltpu.SemaphoreType.DMA,
            pltpu.SemaphoreType.DMA((SEGS,)),
            pltpu.SemaphoreType.DMA,
        ],
    )(body)


def _celu(v):
    return jnp.where(v > 0, v, jnp.exp(jnp.minimum(v, 0.0)) - 1.0)


def _row_spec(width):
    return pl.BlockSpec((BLK, width), lambda i: (i, 0))


def _const_spec(shape):
    return pl.BlockSpec(shape, lambda i: tuple(0 for _ in shape))


def _pad128(v):
    return jnp.concatenate([v, jnp.zeros_like(v)], axis=1)


def _prep1_body(x_ref, dv_ref, y_ref):
    y_ref[...] = _pad128(x_ref[...] * dv_ref[:, :64])


_prep1 = pl.pallas_call(
    _prep1_body,
    grid=(GRID,),
    in_specs=[_row_spec(64), _row_spec(128)],
    out_specs=_row_spec(128),
    out_shape=jax.ShapeDtypeStruct((N_PAD, 128), jnp.float32),
)


def _layer1_body(s_ref, y_ref, dv_ref, w_ref, b_ref, o_ref):
    dv = dv_ref[...]
    z = (s_ref[:, :64] + y_ref[:, :64]) * dv[:, :64]
    h = _celu(jnp.dot(z, w_ref[...], preferred_element_type=jnp.float32)
              + b_ref[...])
    o_ref[...] = h * dv


_layer1 = pl.pallas_call(
    _layer1_body,
    grid=(GRID,),
    in_specs=[_row_spec(128), _row_spec(128), _row_spec(128),
              _const_spec((64, 128)), _const_spec((1, 128))],
    out_specs=_row_spec(128),
    out_shape=jax.ShapeDtypeStruct((N_PAD, 128), jnp.float32),
)


def _layer2a_body(s_ref, y_ref, dv_ref, w2a_ref, o_ref):
    z01 = (s_ref[:, :64] + y_ref[:, :64]) * dv_ref[:, :64]
    o_ref[...] = jnp.dot(z01, w2a_ref[...],
                         preferred_element_type=jnp.float32)


_layer2a = pl.pallas_call(
    _layer2a_body,
    grid=(GRID,),
    in_specs=[_row_spec(128), _row_spec(128), _row_spec(128),
              _const_spec((64, 128))],
    out_specs=_row_spec(128),
    out_shape=jax.ShapeDtypeStruct((N_PAD, 128), jnp.float32),
)


def _layer2b_body(s_ref, y_ref, dv_ref, u_ref, w2b_ref, b2_ref, w3_ref,
                  o_ref):
    dv = dv_ref[...]
    z23 = (s_ref[:, :64] + y_ref[:, 64:]) * dv[:, 64:]
    h2 = _celu(u_ref[...]
               + jnp.dot(z23, w2b_ref[...], preferred_element_type=jnp.float32)
               + b2_ref[...])
    t = jnp.dot(h2, w3_ref[...], preferred_element_type=jnp.float32)
    o_ref[...] = _pad128(t * dv[:, :64])


_layer2b = pl.pallas_call(
    _layer2b_body,
    grid=(GRID,),
    in_specs=[_row_spec(128), _row_spec(128), _row_spec(128), _row_spec(128),
              _const_spec((64, 128)), _const_spec((1, 128)),
              _const_spec((128, 64))],
    out_specs=_row_spec(128),
    out_shape=jax.ShapeDtypeStruct((N_PAD, 128), jnp.float32),
)


def _final_body(s_ref, y_ref, dv_ref, b3_ref, o_ref):
    z = (s_ref[:, :64] + y_ref[:, :64]) * dv_ref[:, :64]
    o_ref[...] = _celu(z + b3_ref[...])


_final = pl.pallas_call(
    _final_body,
    grid=(GRID,),
    in_specs=[_row_spec(128), _row_spec(128), _row_spec(128),
              _const_spec((1, 64))],
    out_specs=_row_spec(64),
    out_shape=jax.ShapeDtypeStruct((N_NODES, 64), jnp.float32),
)


def kernel(x, edge_index, W1, b1, W2, b2, W3, b3):
    pad = E_PAD - E_EDGES
    src = jnp.concatenate([edge_index[0], jnp.zeros((pad,), jnp.int32)])
    dst = jnp.concatenate([edge_index[1],
                           jnp.full((pad,), N_NODES, jnp.int32)])
    src_sp = (src * 4).reshape(NS, SPMM_STREAMS, CHUNK)
    dst_sp = dst.reshape(NS, SPMM_STREAMS, CHUNK)

    def spmm(rounds, y, base=0):
        return _make_spmm(rounds, base)(y.reshape(N4, 32), src_sp, dst_sp)

    dv = _dinv_kernel()(dst_sp)

    y1 = _prep1(x, dv)
    s1 = spmm(1, y1)
    y2 = _layer1(s1, y1, dv, W1, b1.reshape(1, 128))
    s2a = spmm(1, y2)
    s2b = spmm(1, y2, base=2)
    u = _layer2a(s2a, y2, dv, W2[:64])
    y3 = _layer2b(s2b, y2, dv, u, W2[64:], b2.reshape(1, 128), W3)
    s3 = spmm(1, y3)
    return _final(s3, y3, dv, b3.reshape(1, 64))


# speedup vs baseline: 1.0086x; 1.0086x over previous
import functools

import jax
import jax.numpy as jnp
from jax import lax
from jax.experimental import pallas as pl
from jax.experimental.pallas import tpu as pltpu
from jax.experimental.pallas import tpu_sc as plsc

N_NODES = 50000
E_EDGES = 800000

NC = 2
NS = 16
CHUNK = 128

E_PAD = 802816
SPMM_STREAMS = E_PAD // NS // CHUNK

N_PAD = 50176
STRIPE = N_PAD // NS
ZROWS = 32
N4 = 4 * N_PAD

SEGS = 4
BLK = 1024
GRID = N_PAD // BLK


_SC_MESH = dict(core_axis_name="c", subcore_axis_name="s",
                num_cores=NC, num_subcores=NS)
_SC_PARAMS = pltpu.CompilerParams(use_tc_tiling_on_sc=False,
                                  needs_layout_passes=False)


def _fill(buf, rows, width, value):
    def body(i, carry):
        for w0 in range(0, width, 16):
            buf[i, pl.ds(w0, 16)] = jnp.full((16,), value, jnp.float32)
        return carry
    lax.fori_loop(0, rows, body, 0)


def _rsqrt16(v):
    i = plsc.bitcast(v, jnp.int32)
    i = 0x5F3759DF - lax.shift_right_logical(i, 1)
    y = plsc.bitcast(i, jnp.float32)
    for _ in range(3):
        y = y * (1.5 - 0.5 * v * y * y)
    return y


def _dinv_kernel_body(dst_hbm, dv_hbm, acc, dst_st, onesv, cbuf, obuf, ssem,
                      wsem):
    c = lax.axis_index("c")
    s = lax.axis_index("s")
    stripe0 = s * STRIPE
    nmac = SPMM_STREAMS // SEGS

    _fill(onesv, CHUNK, 16, 1.0)
    def idma(k, carry):
        pltpu.sync_copy(onesv, acc.at[pl.ds(stripe0 + k * CHUNK, CHUNK)])
        return carry
    lax.fori_loop(0, STRIPE // CHUNK, idma, 0)
    rem = STRIPE % CHUNK
    if rem:
        pltpu.sync_copy(onesv.at[pl.ds(0, rem)],
                        acc.at[pl.ds(stripe0 + STRIPE - rem, rem)])
    plsc.subcore_barrier()

    def stage(d, par):
        pltpu.async_copy(dst_hbm.at[s, pl.ds(d * SEGS, SEGS)],
                         dst_st.at[par], ssem)

    def stage_wait(d, par):
        pltpu.make_async_copy(dst_hbm.at[s, pl.ds(d * SEGS, SEGS)],
                              dst_st.at[par], ssem).wait()

    stage(0, 0)

    def step(d, carry):
        par = lax.rem(d, 2)
        stage_wait(d, par)

        @pl.when(d > 0)
        def _():
            pltpu.make_async_copy(onesv, acc.at[dst_st.at[par, 0]],
                                  wsem).wait()

        @pl.when(d + 1 < nmac)
        def _():
            stage(d + 1, 1 - par)

        for j in range(SEGS):
            if j > 0:
                pltpu.make_async_copy(onesv, acc.at[dst_st.at[par, 0]],
                                      wsem).wait()
            pltpu.async_copy(onesv, acc.at[dst_st.at[par, j]], wsem, add=True)
        return carry
    lax.fori_loop(0, nmac, step, 0)
    pltpu.make_async_copy(onesv, acc.at[pl.ds(0, CHUNK)], wsem).wait()
    plsc.subcore_barrier()

    half = STRIPE // NC
    base = stripe0 + c * half

    def flush(k, carry):
        pltpu.sync_copy(acc.at[pl.ds(base + k * 32, 32)], cbuf)
        for j in range(32):
            v = _rsqrt16(cbuf[j, pl.ds(0, 16)])
            for t in range(8):
                obuf[j, pl.ds(16 * t, 16)] = v
        pltpu.sync_copy(obuf, dv_hbm.at[pl.ds(base + k * 32, 32)])
        return carry
    lax.fori_loop(0, half // 32, flush, 0)


@functools.lru_cache(maxsize=None)
def _dinv_kernel():
    return functools.partial(
        pl.kernel,
        out_type=jax.ShapeDtypeStruct((N_PAD, 128), jnp.float32),
        mesh=plsc.VectorSubcoreMesh(**_SC_MESH),
        compiler_params=_SC_PARAMS,
        scratch_types=[
            pltpu.VMEM_SHARED((N_PAD, 16), jnp.float32),
            pltpu.VMEM((2, SEGS, CHUNK), jnp.int32),
            pltpu.VMEM((CHUNK, 16), jnp.float32),
            pltpu.VMEM((32, 16), jnp.float32),
            pltpu.VMEM((32, 128), jnp.float32),
            pltpu.SemaphoreType.DMA,
            pltpu.SemaphoreType.DMA,
        ],
    )(_dinv_kernel_body)


@functools.lru_cache(maxsize=None)
def _make_spmm(rounds, base=0):
    nmac = SPMM_STREAMS // SEGS

    def body(y4_hbm, src_hbm, dst_hbm, out_hbm,
             acc, src_st, dst_st, rows, zbuf, ssem, gsem, wsem):
        c = lax.axis_index("c")
        s = lax.axis_index("s")
        stripe0 = s * STRIPE

        _fill(zbuf, ZROWS, 32, 0.0)

        def stage(d, par):
            pltpu.async_copy(src_hbm.at[s, pl.ds(d * SEGS, SEGS)],
                             src_st.at[par], ssem)
            pltpu.async_copy(dst_hbm.at[s, pl.ds(d * SEGS, SEGS)],
                             dst_st.at[par], ssem)

        def stage_wait(d, par):
            pltpu.make_async_copy(src_hbm.at[s, pl.ds(d * SEGS, SEGS)],
                                  src_st.at[par], ssem).wait()
            pltpu.make_async_copy(dst_hbm.at[s, pl.ds(d * SEGS, SEGS)],
                                  dst_st.at[par], ssem).wait()

        def run_round(slab, last):
            col0 = 32 * slab
            table = y4_hbm.at[pl.ds(base + slab, N4 - 3 - base)]

            def zdma(k, carry):
                pltpu.sync_copy(zbuf, acc.at[pl.ds(stripe0 + k * ZROWS, ZROWS)])
                return carry
            lax.fori_loop(0, STRIPE // ZROWS, zdma, 0)
            plsc.subcore_barrier()

            stage(0, 0)

            def step(d, carry):
                par = lax.rem(d, 2)
                stage_wait(d, par)

                @pl.when(d > 0)
                def _():
                    pltpu.make_async_copy(
                        rows.at[0], acc.at[dst_st.at[par, 0]], wsem).wait()

                @pl.when(d + 1 < nmac)
                def _():
                    stage(d + 1, 1 - par)

                for j in range(SEGS):
                    pltpu.async_copy(table.at[src_st.at[par, j]],
                                     rows.at[j], gsem.at[j])
                for j in range(SEGS):
                    pltpu.make_async_copy(table.at[src_st.at[par, j]],
                                          rows.at[j], gsem.at[j]).wait()
                    if j > 0:
                        pltpu.make_async_copy(
                            rows.at[0], acc.at[dst_st.at[par, 0]], wsem).wait()
                    pltpu.async_copy(rows.at[j], acc.at[dst_st.at[par, j]],
                                     wsem, add=True)
                return carry
            lax.fori_loop(0, nmac, step, 0)

            pltpu.make_async_copy(rows.at[0], acc.at[pl.ds(0, CHUNK)],
                                  wsem).wait()
            plsc.subcore_barrier()

            pltpu.sync_copy(acc.at[pl.ds(stripe0, STRIPE)],
                            out_hbm.at[pl.ds(stripe0, STRIPE),
                                       pl.ds(col0, 32)])
            if not last:
                plsc.subcore_barrier()

        for r in range(rounds):
            for cc in range(NC):
                @pl.when(c == cc)
                def _(r=r, cc=cc):
                    run_round(2 * r + cc, r + 1 == rounds)

    return functools.partial(
        pl.kernel,
        out_type=jax.ShapeDtypeStruct((N_PAD, 128), jnp.float32),
        mesh=plsc.VectorSubcoreMesh(**_SC_MESH),
        compiler_params=_SC_PARAMS,
        scratch_types=[
            pltpu.VMEM_SHARED((N_PAD, 32), jnp.float32),
            pltpu.VMEM((2, SEGS, CHUNK), jnp.int32),
            pltpu.VMEM((2, SEGS, CHUNK), jnp.int32),
            pltpu.VMEM((SEGS, CHUNK, 32), jnp.float32),
            pltpu.VMEM((ZROWS, 32), jnp.float32),
            pltpu.SemaphoreType.DMA,
            pltpu.SemaphoreType.DMA((SEGS,)),
            pltpu.SemaphoreType.DMA,
        ],
    )(body)


def _celu(v):
    return jnp.where(v > 0, v, jnp.exp(jnp.minimum(v, 0.0)) - 1.0)


def _row_spec(width):
    return pl.BlockSpec((BLK, width), lambda i: (i, 0))


def _const_spec(shape):
    return pl.BlockSpec(shape, lambda i: tuple(0 for _ in shape))


def _pad128(v):
    return jnp.concatenate([v, jnp.zeros_like(v)], axis=1)


def _prep1_body(x_ref, dv_ref, y_ref):
    y_ref[:, :64] = x_ref[...] * dv_ref[:, :64]


_prep1 = pl.pallas_call(
    _prep1_body,
    grid=(GRID,),
    in_specs=[_row_spec(64), _row_spec(128)],
    out_specs=_row_spec(128),
    out_shape=jax.ShapeDtypeStruct((N_PAD, 128), jnp.float32),
)


def _layer1_body(s_ref, y_ref, dv_ref, w_ref, b_ref, o_ref):
    dv = dv_ref[...]
    z = (s_ref[:, :64] + y_ref[:, :64]) * dv[:, :64]
    h = _celu(jnp.dot(z, w_ref[...], preferred_element_type=jnp.float32)
              + b_ref[...])
    o_ref[...] = h * dv


_layer1 = pl.pallas_call(
    _layer1_body,
    grid=(GRID,),
    in_specs=[_row_spec(128), _row_spec(128), _row_spec(128),
              _const_spec((64, 128)), _const_spec((1, 128))],
    out_specs=_row_spec(128),
    out_shape=jax.ShapeDtypeStruct((N_PAD, 128), jnp.float32),
)


def _layer23_body(s_ref, y_ref, dv_ref, w2_ref, b2_ref, w3_ref, o_ref):
    dv = dv_ref[...]
    z = (s_ref[...] + y_ref[...]) * dv
    h2 = _celu(jnp.dot(z, w2_ref[...], preferred_element_type=jnp.float32)
               + b2_ref[...])
    t = jnp.dot(h2, w3_ref[...], preferred_element_type=jnp.float32)
    o_ref[:, :64] = t * dv[:, :64]


_layer23 = pl.pallas_call(
    _layer23_body,
    grid=(GRID,),
    in_specs=[_row_spec(128), _row_spec(128), _row_spec(128),
              _const_spec((128, 128)), _const_spec((1, 128)),
              _const_spec((128, 64))],
    out_specs=_row_spec(128),
    out_shape=jax.ShapeDtypeStruct((N_PAD, 128), jnp.float32),
)


def _final_body(s_ref, y_ref, dv_ref, b3_ref, o_ref):
    z = (s_ref[:, :64] + y_ref[:, :64]) * dv_ref[:, :64]
    o_ref[...] = _celu(z + b3_ref[...])


_final = pl.pallas_call(
    _final_body,
    grid=(GRID,),
    in_specs=[_row_spec(128), _row_spec(128), _row_spec(128),
              _const_spec((1, 64))],
    out_specs=_row_spec(64),
    out_shape=jax.ShapeDtypeStruct((N_NODES, 64), jnp.float32),
)


def kernel(x, edge_index, W1, b1, W2, b2, W3, b3):
    padrows = (E_PAD - E_EDGES) // CHUNK
    src = jnp.concatenate(
        [(edge_index[0] * 4).reshape(E_EDGES // CHUNK, CHUNK),
         jnp.zeros((padrows, CHUNK), jnp.int32)])
    dst = jnp.concatenate(
        [edge_index[1].reshape(E_EDGES // CHUNK, CHUNK),
         jnp.full((padrows, CHUNK), N_NODES, jnp.int32)])
    src_sp = src.reshape(NS, SPMM_STREAMS, CHUNK)
    dst_sp = dst.reshape(NS, SPMM_STREAMS, CHUNK)

    def spmm(rounds, y, base=0):
        return _make_spmm(rounds, base)(y.reshape(N4, 32), src_sp, dst_sp)

    dv = _dinv_kernel()(dst_sp)

    y1 = _prep1(x, dv)
    s1 = spmm(1, y1)
    y2 = _layer1(s1, y1, dv, W1, b1.reshape(1, 128))
    s2 = spmm(2, y2)
    y3 = _layer23(s2, y2, dv, W2, b2.reshape(1, 128), W3)
    s3 = spmm(1, y3)
    return _final(s3, y3, dv, b3.reshape(1, 64))


# speedup vs baseline: 1.0223x; 1.0135x over previous
import functools

import jax
import jax.numpy as jnp
from jax import lax
from jax.experimental import pallas as pl
from jax.experimental.pallas import tpu as pltpu
from jax.experimental.pallas import tpu_sc as plsc

N_NODES = 50000
E_EDGES = 800000

NC = 2
NS = 16
CHUNK = 128

E_PAD = 802816
SPMM_STREAMS = E_PAD // NS // CHUNK

N_PAD = 50176
STRIPE = N_PAD // NS
ZROWS = 256
N4 = 4 * N_PAD

SEGS = 4
BLK = 1024
GRID = N_PAD // BLK


_SC_MESH = dict(core_axis_name="c", subcore_axis_name="s",
                num_cores=NC, num_subcores=NS)
_SC_PARAMS = pltpu.CompilerParams(use_tc_tiling_on_sc=False,
                                  needs_layout_passes=False)


def _fill(buf, rows, width, value):
    def body(i, carry):
        for w0 in range(0, width, 16):
            buf[i, pl.ds(w0, 16)] = jnp.full((16,), value, jnp.float32)
        return carry
    lax.fori_loop(0, rows, body, 0)


def _rsqrt16(v):
    i = plsc.bitcast(v, jnp.int32)
    i = 0x5F3759DF - lax.shift_right_logical(i, 1)
    y = plsc.bitcast(i, jnp.float32)
    for _ in range(3):
        y = y * (1.5 - 0.5 * v * y * y)
    return y


def _dinv_kernel_body(dst_hbm, dv_hbm, acc, dst_st, onesv, cbuf, obuf, ssem,
                      wsem):
    c = lax.axis_index("c")
    s = lax.axis_index("s")
    stripe0 = s * STRIPE
    nmac = SPMM_STREAMS // SEGS

    _fill(onesv, CHUNK, 16, 1.0)
    def idma(k, carry):
        pltpu.sync_copy(onesv, acc.at[pl.ds(stripe0 + k * CHUNK, CHUNK)])
        return carry
    lax.fori_loop(0, STRIPE // CHUNK, idma, 0)
    rem = STRIPE % CHUNK
    if rem:
        pltpu.sync_copy(onesv.at[pl.ds(0, rem)],
                        acc.at[pl.ds(stripe0 + STRIPE - rem, rem)])
    plsc.subcore_barrier()

    def stage(d, par):
        pltpu.async_copy(dst_hbm.at[s, pl.ds(d * SEGS, SEGS)],
                         dst_st.at[par], ssem)

    def stage_wait(d, par):
        pltpu.make_async_copy(dst_hbm.at[s, pl.ds(d * SEGS, SEGS)],
                              dst_st.at[par], ssem).wait()

    stage(0, 0)

    def step(d, carry):
        par = lax.rem(d, 2)
        stage_wait(d, par)

        @pl.when(d > 0)
        def _():
            pltpu.make_async_copy(onesv, acc.at[dst_st.at[par, 0]],
                                  wsem).wait()

        @pl.when(d + 1 < nmac)
        def _():
            stage(d + 1, 1 - par)

        for j in range(SEGS):
            if j > 0:
                pltpu.make_async_copy(onesv, acc.at[dst_st.at[par, 0]],
                                      wsem).wait()
            pltpu.async_copy(onesv, acc.at[dst_st.at[par, j]], wsem, add=True)
        return carry
    lax.fori_loop(0, nmac, step, 0)
    pltpu.make_async_copy(onesv, acc.at[pl.ds(0, CHUNK)], wsem).wait()
    plsc.subcore_barrier()

    half = STRIPE // NC
    base = stripe0 + c * half

    def flush(k, carry):
        pltpu.sync_copy(acc.at[pl.ds(base + k * 32, 32)], cbuf)
        for j in range(32):
            v = _rsqrt16(cbuf[j, pl.ds(0, 16)])
            for t in range(8):
                obuf[j, pl.ds(16 * t, 16)] = v
        pltpu.sync_copy(obuf, dv_hbm.at[pl.ds(base + k * 32, 32)])
        return carry
    lax.fori_loop(0, half // 32, flush, 0)


@functools.lru_cache(maxsize=None)
def _dinv_kernel():
    return functools.partial(
        pl.kernel,
        out_type=jax.ShapeDtypeStruct((N_PAD, 128), jnp.float32),
        mesh=plsc.VectorSubcoreMesh(**_SC_MESH),
        compiler_params=_SC_PARAMS,
        scratch_types=[
            pltpu.VMEM_SHARED((N_PAD, 16), jnp.float32),
            pltpu.VMEM((2, SEGS, CHUNK), jnp.int32),
            pltpu.VMEM((CHUNK, 16), jnp.float32),
            pltpu.VMEM((32, 16), jnp.float32),
            pltpu.VMEM((32, 128), jnp.float32),
            pltpu.SemaphoreType.DMA,
            pltpu.SemaphoreType.DMA,
        ],
    )(_dinv_kernel_body)


@functools.lru_cache(maxsize=None)
def _make_spmm(rounds, base=0):
    nmac = SPMM_STREAMS // SEGS

    def body(y4_hbm, src_hbm, dst_hbm, out_hbm,
             acc, src_st, dst_st, rows, zbuf, ssem, gsem, wsem, zsem):
        c = lax.axis_index("c")
        s = lax.axis_index("s")
        stripe0 = s * STRIPE

        _fill(zbuf, ZROWS, 32, 0.0)

        def stage(d, par):
            pltpu.async_copy(src_hbm.at[s, pl.ds(d * SEGS, SEGS)],
                             src_st.at[par], ssem)
            pltpu.async_copy(dst_hbm.at[s, pl.ds(d * SEGS, SEGS)],
                             dst_st.at[par], ssem)

        def stage_wait(d, par):
            pltpu.make_async_copy(src_hbm.at[s, pl.ds(d * SEGS, SEGS)],
                                  src_st.at[par], ssem).wait()
            pltpu.make_async_copy(dst_hbm.at[s, pl.ds(d * SEGS, SEGS)],
                                  dst_st.at[par], ssem).wait()

        def run_round(slab, last):
            col0 = 32 * slab
            table = y4_hbm.at[pl.ds(base + slab, N4 - 3 - base)]

            stage(0, 0)
            zfull = STRIPE // ZROWS
            zrem = STRIPE % ZROWS

            def zdma(k, carry):
                pltpu.async_copy(zbuf,
                                 acc.at[pl.ds(stripe0 + k * ZROWS, ZROWS)],
                                 zsem)
                return carry
            lax.fori_loop(0, zfull, zdma, 0)
            if zrem:
                pltpu.async_copy(
                    zbuf.at[pl.ds(0, zrem)],
                    acc.at[pl.ds(stripe0 + zfull * ZROWS, zrem)], zsem)

            def zwait(k, carry):
                pltpu.make_async_copy(
                    zbuf, acc.at[pl.ds(stripe0 + k * ZROWS, ZROWS)],
                    zsem).wait()
                return carry
            lax.fori_loop(0, zfull, zwait, 0)
            if zrem:
                pltpu.make_async_copy(
                    zbuf.at[pl.ds(0, zrem)],
                    acc.at[pl.ds(stripe0 + zfull * ZROWS, zrem)], zsem).wait()
            plsc.subcore_barrier()

            def step(d, carry):
                par = lax.rem(d, 2)
                stage_wait(d, par)

                @pl.when(d > 0)
                def _():
                    pltpu.make_async_copy(
                        rows.at[0], acc.at[dst_st.at[par, 0]], wsem).wait()

                @pl.when(d + 1 < nmac)
                def _():
                    stage(d + 1, 1 - par)

                for j in range(SEGS):
                    pltpu.async_copy(table.at[src_st.at[par, j]],
                                     rows.at[j], gsem.at[j])
                for j in range(SEGS):
                    pltpu.make_async_copy(table.at[src_st.at[par, j]],
                                          rows.at[j], gsem.at[j]).wait()
                    if j > 0:
                        pltpu.make_async_copy(
                            rows.at[0], acc.at[dst_st.at[par, 0]], wsem).wait()
                    pltpu.async_copy(rows.at[j], acc.at[dst_st.at[par, j]],
                                     wsem, add=True)
                return carry
            lax.fori_loop(0, nmac, step, 0)

            pltpu.make_async_copy(rows.at[0], acc.at[pl.ds(0, CHUNK)],
                                  wsem).wait()
            plsc.subcore_barrier()

            pltpu.sync_copy(acc.at[pl.ds(stripe0, STRIPE)],
                            out_hbm.at[pl.ds(stripe0, STRIPE),
                                       pl.ds(col0, 32)])
            if not last:
                plsc.subcore_barrier()

        for r in range(rounds):
            for cc in range(NC):
                @pl.when(c == cc)
                def _(r=r, cc=cc):
                    run_round(2 * r + cc, r + 1 == rounds)

    return functools.partial(
        pl.kernel,
        out_type=jax.ShapeDtypeStruct((N_PAD, 128), jnp.float32),
        mesh=plsc.VectorSubcoreMesh(**_SC_MESH),
        compiler_params=_SC_PARAMS,
        scratch_types=[
            pltpu.VMEM_SHARED((N_PAD, 32), jnp.float32),
            pltpu.VMEM((2, SEGS, CHUNK), jnp.int32),
            pltpu.VMEM((2, SEGS, CHUNK), jnp.int32),
            pltpu.VMEM((SEGS, CHUNK, 32), jnp.float32),
            pltpu.VMEM((ZROWS, 32), jnp.float32),
            pltpu.SemaphoreType.DMA,
            pltpu.SemaphoreType.DMA((SEGS,)),
            pltpu.SemaphoreType.DMA,
            pltpu.SemaphoreType.DMA,
        ],
    )(body)


def _celu(v):
    return jnp.where(v > 0, v, jnp.exp(jnp.minimum(v, 0.0)) - 1.0)


def _row_spec(width):
    return pl.BlockSpec((BLK, width), lambda i: (i, 0))


def _const_spec(shape):
    return pl.BlockSpec(shape, lambda i: tuple(0 for _ in shape))


def _pad128(v):
    return jnp.concatenate([v, jnp.zeros_like(v)], axis=1)


def _prep1_body(x_ref, dv_ref, y_ref):
    y_ref[:, :64] = x_ref[...] * dv_ref[:, :64]


_prep1 = pl.pallas_call(
    _prep1_body,
    grid=(GRID,),
    in_specs=[_row_spec(64), _row_spec(128)],
    out_specs=_row_spec(128),
    out_shape=jax.ShapeDtypeStruct((N_PAD, 128), jnp.float32),
)


def _layer1_body(s_ref, y_ref, dv_ref, w_ref, b_ref, o_ref):
    dv = dv_ref[...]
    z = (s_ref[:, :64] + y_ref[:, :64]) * dv[:, :64]
    h = _celu(jnp.dot(z, w_ref[...], preferred_element_type=jnp.float32)
              + b_ref[...])
    o_ref[...] = h * dv


_layer1 = pl.pallas_call(
    _layer1_body,
    grid=(GRID,),
    in_specs=[_row_spec(128), _row_spec(128), _row_spec(128),
              _const_spec((64, 128)), _const_spec((1, 128))],
    out_specs=_row_spec(128),
    out_shape=jax.ShapeDtypeStruct((N_PAD, 128), jnp.float32),
)


def _layer23_body(s_ref, y_ref, dv_ref, w2_ref, b2_ref, w3_ref, o_ref):
    dv = dv_ref[...]
    z = (s_ref[...] + y_ref[...]) * dv
    h2 = _celu(jnp.dot(z, w2_ref[...], preferred_element_type=jnp.float32)
               + b2_ref[...])
    t = jnp.dot(h2, w3_ref[...], preferred_element_type=jnp.float32)
    o_ref[:, :64] = t * dv[:, :64]


_layer23 = pl.pallas_call(
    _layer23_body,
    grid=(GRID,),
    in_specs=[_row_spec(128), _row_spec(128), _row_spec(128),
              _const_spec((128, 128)), _const_spec((1, 128)),
              _const_spec((128, 64))],
    out_specs=_row_spec(128),
    out_shape=jax.ShapeDtypeStruct((N_PAD, 128), jnp.float32),
)


def _final_body(s_ref, y_ref, dv_ref, b3_ref, o_ref):
    z = (s_ref[:, :64] + y_ref[:, :64]) * dv_ref[:, :64]
    o_ref[...] = _celu(z + b3_ref[...])


_final = pl.pallas_call(
    _final_body,
    grid=(GRID,),
    in_specs=[_row_spec(128), _row_spec(128), _row_spec(128),
              _const_spec((1, 64))],
    out_specs=_row_spec(64),
    out_shape=jax.ShapeDtypeStruct((N_NODES, 64), jnp.float32),
)


def kernel(x, edge_index, W1, b1, W2, b2, W3, b3):
    padrows = (E_PAD - E_EDGES) // CHUNK
    src = jnp.concatenate(
        [(edge_index[0] * 4).reshape(E_EDGES // CHUNK, CHUNK),
         jnp.zeros((padrows, CHUNK), jnp.int32)])
    dst = jnp.concatenate(
        [edge_index[1].reshape(E_EDGES // CHUNK, CHUNK),
         jnp.full((padrows, CHUNK), N_NODES, jnp.int32)])
    src_sp = src.reshape(NS, SPMM_STREAMS, CHUNK)
    dst_sp = dst.reshape(NS, SPMM_STREAMS, CHUNK)

    def spmm(rounds, y, base=0):
        return _make_spmm(rounds, base)(y.reshape(N4, 32), src_sp, dst_sp)

    dv = _dinv_kernel()(dst_sp)

    y1 = _prep1(x, dv)
    s1 = spmm(1, y1)
    y2 = _layer1(s1, y1, dv, W1, b1.reshape(1, 128))
    s2 = spmm(2, y2)
    y3 = _layer23(s2, y2, dv, W2, b2.reshape(1, 128), W3)
    s3 = spmm(1, y3)
    return _final(s3, y3, dv, b3.reshape(1, 64))
